# baseline (speedup 1.0000x reference)
import functools
import math

import jax
import jax.numpy as jnp
from jax import lax
from jax.experimental import pallas as pl
from jax.experimental.pallas import tpu as pltpu
from jax.experimental.pallas import tpu_sc as plsc

VOCAB = 1000000
VOCABP = 1000064
DIM = 64
LANES = 16
PW = 128
SCALE = math.sqrt(DIM)

_info = plsc.get_sparse_core_info()
_NC = _info.num_cores
_NW = _NC * _info.num_subcores

_mesh = plsc.VectorSubcoreMesh(core_axis_name="c", subcore_axis_name="s")
_params = pltpu.CompilerParams(
    use_tc_tiling_on_sc=True, needs_layout_passes=False
)


def _wid():
    return lax.axis_index("s") * _NC + lax.axis_index("c")


@functools.partial(
    pl.kernel,
    mesh=_mesh,
    out_type=jax.ShapeDtypeStruct((VOCABP // 2, PW), jnp.float32),
    scratch_types=[
        pltpu.VMEM((DIM, 128), jnp.float32),
        pltpu.VMEM((64, PW), jnp.float32),
    ],
    compiler_params=_params,
)
def _pack_pairs(tt_hbm, p_hbm, blk_v, ps_v):
    wid = _wid()
    iota = lax.iota(jnp.int32, LANES)
    n_full = VOCABP // 128

    def do_block(i0, q_rows, blk, ps):
        def q_body(q, c):
            for h in range(2):
                col = jnp.full((LANES,), 2 * q + h, dtype=jnp.int32)
                for dg in range(DIM // LANES):
                    rows = iota + (dg * LANES)
                    val = plsc.load_gather(blk, [rows, col])
                    ps[q, pl.ds(h * DIM + dg * LANES, LANES)] = val
            return c

        lax.fori_loop(0, q_rows, q_body, 0)

    def blk_body(k, carry):
        bid = wid + _NW * k

        @pl.when(bid < n_full)
        def _():
            i0 = pl.multiple_of(bid * 128, 128)
            pltpu.sync_copy(tt_hbm.at[:, pl.ds(i0, 128)], blk_v)
            do_block(i0, 64, blk_v, ps_v)
            pltpu.sync_copy(ps_v, p_hbm.at[pl.ds(bid * 64, 64)])

        return carry

    n_iter = (n_full + _NW - 1) // _NW
    lax.fori_loop(0, n_iter, blk_body, 0)


@functools.lru_cache(maxsize=None)
def _make_lookup(BA, T):
    nb = BA // 128
    per_w = nb // _NW if nb >= _NW else 0
    assert nb % _NW == 0

    @functools.partial(
        pl.kernel,
        mesh=_mesh,
        out_type=jax.ShapeDtypeStruct((T, DIM, BA), jnp.float32),
        scratch_types=[
            pltpu.VMEM((128,), jnp.int32),
            pltpu.VMEM((128,), jnp.int32),
            pltpu.VMEM((128,), jnp.int32),
            pltpu.VMEM((128, PW), jnp.float32),
            pltpu.VMEM((DIM, 128), jnp.float32),
            pltpu.SemaphoreType.DMA,
        ],
        compiler_params=_params,
    )
    def _lookup(idxt_hbm, p_hbm, out_hbm, ib_v, gb_v, hb_v, rows_v, sb_v, sem):
        wid = _wid()
        iota = lax.iota(jnp.int32, LANES)

        def tile(bblk, t):
            b0 = pl.multiple_of(bblk * 128, 128)
            pltpu.sync_copy(idxt_hbm.at[t, pl.ds(b0, 128)], ib_v)
            for g in range(8):
                s = pl.ds(g * LANES, LANES)
                v = ib_v[s]
                gb_v[s] = lax.shift_right_logical(v, 1)
                hb_v[s] = lax.shift_left(jnp.bitwise_and(v, 1), 6)
            pltpu.async_copy(p_hbm.at[gb_v], rows_v, sem).wait()

            def d_body(d, c):
                for g in range(8):
                    s = pl.ds(g * LANES, LANES)
                    rows = iota + (g * LANES)
                    col = hb_v[s] + d
                    val = plsc.load_gather(rows_v, [rows, col])
                    sb_v[d, s] = val * SCALE
                return c

            lax.fori_loop(0, DIM, d_body, 0)
            pltpu.sync_copy(sb_v, out_hbm.at[t, :, pl.ds(b0, 128)])

        def b_body(k, carry):
            bblk = wid + _NW * k

            def t_loop(t, c2):
                tile(bblk, t)
                return c2

            lax.fori_loop(0, T, t_loop, 0)
            return carry

        lax.fori_loop(0, per_w, b_body, 0)

    return _lookup


def kernel(inp, table):
    ba, t = inp.shape
    idxt = inp.T.astype(jnp.int32)
    tt = jnp.pad(table, ((0, VOCABP - VOCAB), (0, 0))).T
    packed = _pack_pairs(tt)
    out3 = _make_lookup(ba, t)(idxt, packed)
    return jnp.transpose(out3, (2, 0, 1))


# speedup vs baseline: 1.5020x; 1.5020x over previous
import functools
import math

import jax
import jax.numpy as jnp
from jax import lax
from jax.experimental import pallas as pl
from jax.experimental.pallas import tpu as pltpu
from jax.experimental.pallas import tpu_sc as plsc

VOCAB = 1000000
DIM = 64
LANES = 16
PW = 128
BK = 512
SCALE = math.sqrt(DIM)

_info = plsc.get_sparse_core_info()
_NC = _info.num_cores
_NW = _NC * _info.num_subcores

_mesh = plsc.VectorSubcoreMesh(core_axis_name="c", subcore_axis_name="s")
_params = pltpu.CompilerParams(
    use_tc_tiling_on_sc=True, needs_layout_passes=False
)


def _pack_body(in_ref, out_ref):
    x = in_ref[...]
    y = jnp.transpose(x)
    out_ref[...] = jnp.concatenate(
        [y, jnp.zeros((BK, PW - DIM), jnp.float32)], axis=1
    )


_pack_tc = pl.pallas_call(
    _pack_body,
    grid=(pl.cdiv(VOCAB, BK),),
    in_specs=[pl.BlockSpec((DIM, BK), lambda i: (0, i))],
    out_specs=pl.BlockSpec((BK, PW), lambda i: (i, 0)),
    out_shape=jax.ShapeDtypeStruct((VOCAB, PW), jnp.float32),
)


@functools.lru_cache(maxsize=None)
def _make_lookup(BA, T):
    nb = BA // 128
    assert nb % _NW == 0
    per_w = nb // _NW

    @functools.partial(
        pl.kernel,
        mesh=_mesh,
        out_type=jax.ShapeDtypeStruct((T, DIM, BA), jnp.float32),
        scratch_types=[
            pltpu.VMEM((128,), jnp.int32),
            pltpu.VMEM((128, PW), jnp.float32),
            pltpu.VMEM((DIM, 128), jnp.float32),
            pltpu.SemaphoreType.DMA,
        ],
        compiler_params=_params,
    )
    def _lookup(idxt_hbm, p_hbm, out_hbm, ib_v, rows_v, sb_v, sem):
        wid = lax.axis_index("s") * _NC + lax.axis_index("c")
        iota = lax.iota(jnp.int32, LANES)
        rowvecs = [iota + (g * LANES) for g in range(8)]

        def tile(bblk, t):
            b0 = pl.multiple_of(bblk * 128, 128)
            pltpu.sync_copy(idxt_hbm.at[t, pl.ds(b0, 128)], ib_v)
            pltpu.async_copy(p_hbm.at[ib_v], rows_v, sem).wait()

            def d_body(k, c):
                for u in range(4):
                    d = k * 4 + u
                    col = jnp.full((LANES,), 0, jnp.int32) + d
                    for g in range(8):
                        val = plsc.load_gather(rows_v, [rowvecs[g], col])
                        sb_v[d, pl.ds(g * LANES, LANES)] = val * SCALE
                return c

            lax.fori_loop(0, DIM // 4, d_body, 0)
            pltpu.sync_copy(sb_v, out_hbm.at[t, :, pl.ds(b0, 128)])

        def b_body(k, carry):
            bblk = wid + _NW * k

            def t_loop(t, c2):
                tile(bblk, t)
                return c2

            lax.fori_loop(0, T, t_loop, 0)
            return carry

        lax.fori_loop(0, per_w, b_body, 0)

    return _lookup


def kernel(inp, table):
    ba, t = inp.shape
    idxt = inp.T.astype(jnp.int32)
    packed = _pack_tc(table.T)
    out3 = _make_lookup(ba, t)(idxt, packed)
    return jnp.transpose(out3, (2, 0, 1))


# speedup vs baseline: 3.2605x; 2.1708x over previous
import functools
import math

import jax
import jax.numpy as jnp
from jax import lax
from jax.experimental import pallas as pl
from jax.experimental.pallas import tpu as pltpu
from jax.experimental.pallas import tpu_sc as plsc

VOCAB = 1000000
DIM = 64
LANES = 16
PW = 128
SCALE = math.sqrt(DIM)

_info = plsc.get_sparse_core_info()
_NC = _info.num_cores
_NW = _NC * _info.num_subcores

_mesh = plsc.VectorSubcoreMesh(core_axis_name="c", subcore_axis_name="s")
_params = pltpu.CompilerParams(
    use_tc_tiling_on_sc=True, needs_layout_passes=False
)


@functools.lru_cache(maxsize=None)
def _make_lookup(BA, T):
    assert BA // 128 == _NW and BA % 128 == 0 and T % 2 == 0

    @functools.partial(
        pl.kernel,
        mesh=_mesh,
        out_type=jax.ShapeDtypeStruct((T, DIM, BA), jnp.float32),
        scratch_types=[
            pltpu.VMEM((128,), jnp.int32),
            pltpu.VMEM((128,), jnp.int32),
            pltpu.VMEM((128, PW), jnp.float32),
            pltpu.VMEM((128, PW), jnp.float32),
            pltpu.VMEM((DIM, 128), jnp.float32),
            pltpu.VMEM((DIM, 128), jnp.float32),
            pltpu.SemaphoreType.DMA,
            pltpu.SemaphoreType.DMA,
            pltpu.SemaphoreType.DMA,
            pltpu.SemaphoreType.DMA,
        ],
        compiler_params=_params,
    )
    def _lookup(
        idxt_hbm, p_hbm, out_hbm,
        ib0, ib1, rows0, rows1, sb0, sb1, gs0, gs1, ws0, ws1,
    ):
        wid = lax.axis_index("s") * _NC + lax.axis_index("c")
        b0 = pl.multiple_of(wid * 128, 128)
        iota = lax.iota(jnp.int32, LANES)
        colgs = [iota + (g * LANES) for g in range(8)]

        def fetch(t, ib, rows, gs):
            pltpu.sync_copy(idxt_hbm.at[t, pl.ds(b0, 128)], ib)
            pltpu.async_copy(p_hbm.at[ib], rows, gs)

        def gwait(rows, gs):
            pltpu.make_async_copy(p_hbm.at[pl.ds(0, 128)], rows, gs).wait()

        def put(t, sb, ws):
            pltpu.async_copy(sb, out_hbm.at[t, :, pl.ds(b0, 128)], ws)

        def wwait(sb, ws):
            pltpu.make_async_copy(
                sb, out_hbm.at[0, :, pl.ds(b0, 128)], ws
            ).wait()

        def transpose(rows, sb):
            def d_body(k, c):
                for u in range(2):
                    d = k * 2 + u
                    rowv = jnp.bitwise_and(iota + d, DIM - 1)
                    for g in range(8):
                        val = plsc.load_gather(rows, [colgs[g], rowv])
                        plsc.store_scatter(sb, [rowv, colgs[g]], val)
                return c

            lax.fori_loop(0, DIM // 2, d_body, 0)

        fetch(0, ib0, rows0, gs0)

        def body(k, carry):
            t = k * 2
            fetch(t + 1, ib1, rows1, gs1)

            @pl.when(k > 0)
            def _():
                wwait(sb0, ws0)

            gwait(rows0, gs0)
            transpose(rows0, sb0)
            put(t, sb0, ws0)

            @pl.when(t + 2 < T)
            def _():
                fetch(t + 2, ib0, rows0, gs0)

            @pl.when(k > 0)
            def _():
                wwait(sb1, ws1)

            gwait(rows1, gs1)
            transpose(rows1, sb1)
            put(t + 1, sb1, ws1)
            return carry

        lax.fori_loop(0, T // 2, body, 0)
        wwait(sb0, ws0)
        wwait(sb1, ws1)

    return _lookup


def kernel(inp, table):
    ba, t = inp.shape
    idxt = inp.T.astype(jnp.int32)
    tpad = jnp.pad(table * jnp.float32(SCALE), ((0, 0), (0, PW - DIM)))
    out3 = _make_lookup(ba, t)(idxt, tpad)
    return jnp.transpose(out3, (2, 0, 1))


# speedup vs baseline: 3.6937x; 1.1329x over previous
import functools
import math

import jax
import jax.numpy as jnp
from jax import lax
from jax.experimental import pallas as pl
from jax.experimental.pallas import tpu as pltpu
from jax.experimental.pallas import tpu_sc as plsc

VOCAB = 1000000
DIM = 64
LANES = 16
PW = 128
SCALE = math.sqrt(DIM)

_info = plsc.get_sparse_core_info()
_NC = _info.num_cores
_NW = _NC * _info.num_subcores

_mesh = plsc.VectorSubcoreMesh(core_axis_name="c", subcore_axis_name="s")
_params = pltpu.CompilerParams(
    use_tc_tiling_on_sc=True, needs_layout_passes=False
)

_PBK = 4096


def _pad_body(x_ref, o_ref):
    o_ref[...] = jnp.concatenate(
        [
            x_ref[...] * jnp.float32(SCALE),
            jnp.zeros((_PBK, PW - DIM), jnp.float32),
        ],
        axis=1,
    )


_pad_tc = pl.pallas_call(
    _pad_body,
    grid=(pl.cdiv(VOCAB, _PBK),),
    in_specs=[pl.BlockSpec((_PBK, DIM), lambda i: (i, 0))],
    out_specs=pl.BlockSpec((_PBK, PW), lambda i: (i, 0)),
    out_shape=jax.ShapeDtypeStruct((VOCAB, PW), jnp.float32),
)


@functools.lru_cache(maxsize=None)
def _make_lookup(BA, T):
    assert BA // 128 == _NW and BA % 128 == 0 and T % 2 == 0

    @functools.partial(
        pl.kernel,
        mesh=_mesh,
        out_type=jax.ShapeDtypeStruct((T, DIM, BA), jnp.float32),
        scratch_types=[
            pltpu.VMEM((128,), jnp.int32),
            pltpu.VMEM((128,), jnp.int32),
            pltpu.VMEM((128, PW), jnp.float32),
            pltpu.VMEM((128, PW), jnp.float32),
            pltpu.VMEM((DIM, 128), jnp.float32),
            pltpu.VMEM((DIM, 128), jnp.float32),
            pltpu.SemaphoreType.DMA,
            pltpu.SemaphoreType.DMA,
            pltpu.SemaphoreType.DMA,
            pltpu.SemaphoreType.DMA,
        ],
        compiler_params=_params,
    )
    def _lookup(
        idxt_hbm, p_hbm, out_hbm,
        ib0, ib1, rows0, rows1, sb0, sb1, gs0, gs1, ws0, ws1,
    ):
        wid = lax.axis_index("s") * _NC + lax.axis_index("c")
        b0 = pl.multiple_of(wid * 128, 128)
        iota = lax.iota(jnp.int32, LANES)
        colgs = [iota + (g * LANES) for g in range(8)]

        def fetch(t, ib, rows, gs):
            pltpu.sync_copy(idxt_hbm.at[t, pl.ds(b0, 128)], ib)
            pltpu.async_copy(p_hbm.at[ib], rows, gs)

        def gwait(rows, gs):
            pltpu.make_async_copy(p_hbm.at[pl.ds(0, 128)], rows, gs).wait()

        def put(t, sb, ws):
            pltpu.async_copy(sb, out_hbm.at[t, :, pl.ds(b0, 128)], ws)

        def wwait(sb, ws):
            pltpu.make_async_copy(
                sb, out_hbm.at[0, :, pl.ds(b0, 128)], ws
            ).wait()

        def transpose(rows, sb):
            def d_body(k, c):
                for u in range(2):
                    d = k * 2 + u
                    rowv = jnp.bitwise_and(iota + d, DIM - 1)
                    for g in range(8):
                        val = plsc.load_gather(rows, [colgs[g], rowv])
                        plsc.store_scatter(sb, [rowv, colgs[g]], val)
                return c

            lax.fori_loop(0, DIM // 2, d_body, 0)

        fetch(0, ib0, rows0, gs0)

        def body(k, carry):
            t = k * 2
            fetch(t + 1, ib1, rows1, gs1)

            @pl.when(k > 0)
            def _():
                wwait(sb0, ws0)

            gwait(rows0, gs0)
            transpose(rows0, sb0)
            put(t, sb0, ws0)

            @pl.when(t + 2 < T)
            def _():
                fetch(t + 2, ib0, rows0, gs0)

            @pl.when(k > 0)
            def _():
                wwait(sb1, ws1)

            gwait(rows1, gs1)
            transpose(rows1, sb1)
            put(t + 1, sb1, ws1)
            return carry

        lax.fori_loop(0, T // 2, body, 0)
        wwait(sb0, ws0)
        wwait(sb1, ws1)

    return _lookup


def kernel(inp, table):
    ba, t = inp.shape
    idxt = inp.T.astype(jnp.int32)
    tpad = _pad_tc(table)
    out3 = _make_lookup(ba, t)(idxt, tpad)
    return jnp.transpose(out3, (2, 0, 1))


# speedup vs baseline: 5.6528x; 1.5304x over previous
import functools
import math

import jax
import jax.numpy as jnp
from jax import lax
from jax.experimental import pallas as pl
from jax.experimental.pallas import tpu as pltpu
from jax.experimental.pallas import tpu_sc as plsc

VOCAB = 1000000
DIM = 64
LANES = 16
PW = 128
SCALE = math.sqrt(DIM)

_info = plsc.get_sparse_core_info()
_NC = _info.num_cores
_NW = _NC * _info.num_subcores

_mesh = plsc.VectorSubcoreMesh(core_axis_name="c", subcore_axis_name="s")
_params = pltpu.CompilerParams(
    use_tc_tiling_on_sc=True, needs_layout_passes=False
)

_PBK = 8192


def _pad_body(x_ref, o_ref):
    y = jnp.transpose(x_ref[...]) * jnp.float32(SCALE)
    o_ref[...] = jnp.concatenate(
        [y, jnp.zeros((_PBK, PW - DIM), jnp.float32)], axis=1
    )


_pad_tc = pl.pallas_call(
    _pad_body,
    grid=(pl.cdiv(VOCAB, _PBK),),
    in_specs=[pl.BlockSpec((DIM, _PBK), lambda i: (0, i))],
    out_specs=pl.BlockSpec((_PBK, PW), lambda i: (i, 0)),
    out_shape=jax.ShapeDtypeStruct((VOCAB, PW), jnp.float32),
)


@functools.lru_cache(maxsize=None)
def _make_lookup(BA, T):
    assert BA // 128 == _NW and BA % 128 == 0 and T % 2 == 0

    @functools.partial(
        pl.kernel,
        mesh=_mesh,
        out_type=jax.ShapeDtypeStruct((T, DIM, BA), jnp.float32),
        scratch_types=[
            pltpu.VMEM((128,), jnp.int32),
            pltpu.VMEM((128,), jnp.int32),
            pltpu.VMEM((128, PW), jnp.float32),
            pltpu.VMEM((128, PW), jnp.float32),
            pltpu.VMEM((DIM, 128), jnp.float32),
            pltpu.VMEM((DIM, 128), jnp.float32),
            pltpu.SemaphoreType.DMA,
            pltpu.SemaphoreType.DMA,
            pltpu.SemaphoreType.DMA,
            pltpu.SemaphoreType.DMA,
        ],
        compiler_params=_params,
    )
    def _lookup(
        idxt_hbm, p_hbm, out_hbm,
        ib0, ib1, rows0, rows1, sb0, sb1, gs0, gs1, ws0, ws1,
    ):
        wid = lax.axis_index("s") * _NC + lax.axis_index("c")
        b0 = pl.multiple_of(wid * 128, 128)
        iota = lax.iota(jnp.int32, LANES)
        colgs = [iota + (g * LANES) for g in range(8)]

        def fetch(t, ib, rows, gs):
            pltpu.sync_copy(idxt_hbm.at[t, pl.ds(b0, 128)], ib)
            pltpu.async_copy(p_hbm.at[ib], rows, gs)

        def gwait(rows, gs):
            pltpu.make_async_copy(p_hbm.at[pl.ds(0, 128)], rows, gs).wait()

        def put(t, sb, ws):
            pltpu.async_copy(sb, out_hbm.at[t, :, pl.ds(b0, 128)], ws)

        def wwait(sb, ws):
            pltpu.make_async_copy(
                sb, out_hbm.at[0, :, pl.ds(b0, 128)], ws
            ).wait()

        def transpose(rows, sb):
            def d_body(k, c):
                for u in range(2):
                    d = k * 2 + u
                    rowv = jnp.bitwise_and(iota + d, DIM - 1)
                    for g in range(8):
                        val = plsc.load_gather(rows, [colgs[g], rowv])
                        plsc.store_scatter(sb, [rowv, colgs[g]], val)
                return c

            lax.fori_loop(0, DIM // 2, d_body, 0)

        fetch(0, ib0, rows0, gs0)

        def body(k, carry):
            t = k * 2
            fetch(t + 1, ib1, rows1, gs1)

            @pl.when(k > 0)
            def _():
                wwait(sb0, ws0)

            gwait(rows0, gs0)
            transpose(rows0, sb0)
            put(t, sb0, ws0)

            @pl.when(t + 2 < T)
            def _():
                fetch(t + 2, ib0, rows0, gs0)

            @pl.when(k > 0)
            def _():
                wwait(sb1, ws1)

            gwait(rows1, gs1)
            transpose(rows1, sb1)
            put(t + 1, sb1, ws1)
            return carry

        lax.fori_loop(0, T // 2, body, 0)
        wwait(sb0, ws0)
        wwait(sb1, ws1)

    return _lookup


def kernel(inp, table):
    ba, t = inp.shape
    idxt = inp.T.astype(jnp.int32)
    tpad = _pad_tc(table.T)
    out3 = _make_lookup(ba, t)(idxt, tpad)
    return jnp.transpose(out3, (2, 0, 1))


# speedup vs baseline: 6.5164x; 1.1528x over previous
import functools
import math

import jax
import jax.numpy as jnp
from jax import lax
from jax.experimental import pallas as pl
from jax.experimental.pallas import tpu as pltpu
from jax.experimental.pallas import tpu_sc as plsc

VOCAB = 1000000
DIM = 64
LANES = 16
PW = 128
SCALE = math.sqrt(DIM)

_info = plsc.get_sparse_core_info()
_NC = _info.num_cores
_NW = _NC * _info.num_subcores

_mesh = plsc.VectorSubcoreMesh(core_axis_name="c", subcore_axis_name="s")
_params = pltpu.CompilerParams(
    use_tc_tiling_on_sc=True, needs_layout_passes=False
)

_PBK = 8192


def _pad_body(x_ref, o_ref):
    y = jnp.transpose(x_ref[...]) * jnp.float32(SCALE)
    o_ref[...] = jnp.concatenate(
        [y, jnp.zeros((_PBK, PW - DIM), jnp.float32)], axis=1
    )


_pad_tc = pl.pallas_call(
    _pad_body,
    grid=(pl.cdiv(VOCAB, _PBK),),
    in_specs=[pl.BlockSpec((DIM, _PBK), lambda i: (0, i))],
    out_specs=pl.BlockSpec((_PBK, PW), lambda i: (i, 0)),
    out_shape=jax.ShapeDtypeStruct((VOCAB, PW), jnp.float32),
)


@functools.lru_cache(maxsize=None)
def _make_lookup(BA, T):
    assert BA // 128 == _NW and BA % 128 == 0 and T % 2 == 0

    @functools.partial(
        pl.kernel,
        mesh=_mesh,
        out_type=jax.ShapeDtypeStruct((T, DIM, BA), jnp.float32),
        scratch_types=[
            pltpu.VMEM((T, 128), jnp.int32),
            pltpu.VMEM((128, PW), jnp.float32),
            pltpu.VMEM((128, PW), jnp.float32),
            pltpu.VMEM((DIM, 128), jnp.float32),
            pltpu.VMEM((DIM, 128), jnp.float32),
            pltpu.SemaphoreType.DMA,
            pltpu.SemaphoreType.DMA,
            pltpu.SemaphoreType.DMA,
            pltpu.SemaphoreType.DMA,
        ],
        compiler_params=_params,
    )
    def _lookup(
        idxt_hbm, p_hbm, out_hbm,
        ibig, rows0, rows1, sb0, sb1, gs0, gs1, ws0, ws1,
    ):
        wid = lax.axis_index("s") * _NC + lax.axis_index("c")
        b0 = pl.multiple_of(wid * 128, 128)
        iota = lax.iota(jnp.int32, LANES)
        colgs = [iota + (g * LANES) for g in range(8)]

        pltpu.sync_copy(idxt_hbm.at[:, pl.ds(b0, 128)], ibig)

        def fetch(t, rows, gs):
            pltpu.async_copy(p_hbm.at[ibig.at[t]], rows, gs)

        def gwait(rows, gs):
            pltpu.make_async_copy(p_hbm.at[pl.ds(0, 128)], rows, gs).wait()

        def put(t, sb, ws):
            pltpu.async_copy(sb, out_hbm.at[t, :, pl.ds(b0, 128)], ws)

        def wwait(sb, ws):
            pltpu.make_async_copy(
                sb, out_hbm.at[0, :, pl.ds(b0, 128)], ws
            ).wait()

        def transpose(rows, sb):
            def d_body(k, c):
                for u in range(2):
                    d = k * 2 + u
                    rowv = jnp.bitwise_and(iota + d, DIM - 1)
                    for g in range(8):
                        val = plsc.load_gather(rows, [colgs[g], rowv])
                        plsc.store_scatter(sb, [rowv, colgs[g]], val)
                return c

            lax.fori_loop(0, DIM // 2, d_body, 0)

        fetch(0, rows0, gs0)

        def body(k, carry):
            t = k * 2
            fetch(t + 1, rows1, gs1)

            @pl.when(k > 0)
            def _():
                wwait(sb0, ws0)

            gwait(rows0, gs0)
            transpose(rows0, sb0)
            put(t, sb0, ws0)

            @pl.when(t + 2 < T)
            def _():
                fetch(t + 2, rows0, gs0)

            @pl.when(k > 0)
            def _():
                wwait(sb1, ws1)

            gwait(rows1, gs1)
            transpose(rows1, sb1)
            put(t + 1, sb1, ws1)
            return carry

        lax.fori_loop(0, T // 2, body, 0)
        wwait(sb0, ws0)
        wwait(sb1, ws1)

    return _lookup


def kernel(inp, table):
    ba, t = inp.shape
    idxt = inp.T.astype(jnp.int32)
    tpad = _pad_tc(table.T)
    out3 = _make_lookup(ba, t)(idxt, tpad)
    return jnp.transpose(out3, (2, 0, 1))


# speedup vs baseline: 6.5449x; 1.0044x over previous
import functools
import math

import jax
import jax.numpy as jnp
from jax import lax
from jax.experimental import pallas as pl
from jax.experimental.pallas import tpu as pltpu
from jax.experimental.pallas import tpu_sc as plsc

VOCAB = 1000000
DIM = 64
LANES = 16
PW = 128
SCALE = math.sqrt(DIM)

_info = plsc.get_sparse_core_info()
_NC = _info.num_cores
_NW = _NC * _info.num_subcores

_mesh = plsc.VectorSubcoreMesh(core_axis_name="c", subcore_axis_name="s")
_params = pltpu.CompilerParams(
    use_tc_tiling_on_sc=True, needs_layout_passes=False
)

_PBK = 8192


def _pad_body(x_ref, o_ref):
    y = jnp.transpose(x_ref[...]) * jnp.float32(SCALE)
    o_ref[...] = jnp.concatenate(
        [y, jnp.zeros((_PBK, PW - DIM), jnp.float32)], axis=1
    )


_pad_tc = pl.pallas_call(
    _pad_body,
    grid=(pl.cdiv(VOCAB, _PBK),),
    in_specs=[pl.BlockSpec((DIM, _PBK), lambda i: (0, i))],
    out_specs=pl.BlockSpec((_PBK, PW), lambda i: (i, 0)),
    out_shape=jax.ShapeDtypeStruct((VOCAB, PW), jnp.float32),
)


@functools.lru_cache(maxsize=None)
def _make_lookup(BA, T):
    assert BA // 128 == _NW and BA % 128 == 0 and T % 4 == 0

    @functools.partial(
        pl.kernel,
        mesh=_mesh,
        out_type=jax.ShapeDtypeStruct((T, DIM, BA), jnp.float32),
        scratch_types=[
            pltpu.VMEM((T, 128), jnp.int32),
            pltpu.VMEM((128, PW), jnp.float32),
            pltpu.VMEM((128, PW), jnp.float32),
            pltpu.VMEM((128, PW), jnp.float32),
            pltpu.VMEM((128, PW), jnp.float32),
            pltpu.VMEM((DIM, 128), jnp.float32),
            pltpu.VMEM((DIM, 128), jnp.float32),
            pltpu.SemaphoreType.DMA,
            pltpu.SemaphoreType.DMA,
            pltpu.SemaphoreType.DMA,
            pltpu.SemaphoreType.DMA,
            pltpu.SemaphoreType.DMA,
            pltpu.SemaphoreType.DMA,
        ],
        compiler_params=_params,
    )
    def _lookup(
        idxt_hbm, p_hbm, out_hbm,
        ibig, rows0, rows1, rows2, rows3, sb0, sb1,
        gs0, gs1, gs2, gs3, ws0, ws1,
    ):
        wid = lax.axis_index("s") * _NC + lax.axis_index("c")
        b0 = pl.multiple_of(wid * 128, 128)
        iota = lax.iota(jnp.int32, LANES)
        colgs = [iota + (g * LANES) for g in range(8)]

        pltpu.sync_copy(idxt_hbm.at[:, pl.ds(b0, 128)], ibig)

        def fetch(t, rows, gs):
            pltpu.async_copy(p_hbm.at[ibig.at[t]], rows, gs)

        def gwait(rows, gs):
            pltpu.make_async_copy(p_hbm.at[pl.ds(0, 128)], rows, gs).wait()

        def put(t, sb, ws):
            pltpu.async_copy(sb, out_hbm.at[t, :, pl.ds(b0, 128)], ws)

        def wwait(sb, ws):
            pltpu.make_async_copy(
                sb, out_hbm.at[0, :, pl.ds(b0, 128)], ws
            ).wait()

        def transpose(rows, sb):
            def d_body(k, c):
                for u in range(4):
                    d = k * 4 + u
                    rowv = jnp.bitwise_and(iota + d, DIM - 1)
                    for g in range(8):
                        val = plsc.load_gather(rows, [colgs[g], rowv])
                        plsc.store_scatter(sb, [rowv, colgs[g]], val)
                return c

            lax.fori_loop(0, DIM // 4, d_body, 0)

        rbufs = [rows0, rows1, rows2, rows3]
        gsems = [gs0, gs1, gs2, gs3]
        sbufs = [sb0, sb1]
        wsems = [ws0, ws1]

        for j in range(3):
            fetch(j, rbufs[j], gsems[j])

        def body(k, carry):
            t = k * 4
            for j in range(4):
                nt = t + j + 3

                @pl.when(nt < T)
                def _(nt=nt, j=j):
                    fetch(nt, rbufs[(j + 3) % 4], gsems[(j + 3) % 4])

                if j < 2:

                    @pl.when(k > 0)
                    def _(j=j):
                        wwait(sbufs[j], wsems[j])

                else:
                    wwait(sbufs[j % 2], wsems[j % 2])
                gwait(rbufs[j], gsems[j])
                transpose(rbufs[j], sbufs[j % 2])
                put(t + j, sbufs[j % 2], wsems[j % 2])
            return carry

        lax.fori_loop(0, T // 4, body, 0)
        wwait(sb0, ws0)
        wwait(sb1, ws1)

    return _lookup


def kernel(inp, table):
    ba, t = inp.shape
    idxt = inp.T.astype(jnp.int32)
    tpad = _pad_tc(table.T)
    out3 = _make_lookup(ba, t)(idxt, tpad)
    return jnp.transpose(out3, (2, 0, 1))
